# baseline (speedup 1.0000x reference)
import math

import jax
import jax.numpy as jnp
from jax import lax
from jax.experimental import pallas as pl
from jax.experimental.pallas import tpu as pltpu
from jax.experimental.pallas import tpu_sc as plsc

DIM = 256
N_EMBED = 8192
NTOK = 9216
ENTROPY_PENALTY = 0.1
BETA = 0.25
EPS = 1e-05
LOG_N = math.log(float(N_EMBED))

TM = 512
KT = 2048
NK = N_EMBED // KT

SC_NC = 2
SC_NS = 16
SC_NW = SC_NC * SC_NS
SC_BPW = NTOK // SC_NW
HIST_PAD = N_EMBED + 16


def _codebook_body(emb_ref, w_ref, b_ref, out_ref, out2_ref):
    mm = lax.dot_general(emb_ref[...], w_ref[...], (((1,), (1,)), ((), ())),
                         preferred_element_type=jnp.float32)
    cb = mm + b_ref[...]
    out_ref[...] = cb
    out2_ref[...] = cb + cb


def _codebook_call(emb, w, b2d, *, interpret=False):
    return pl.pallas_call(
        _codebook_body,
        grid=(N_EMBED // 512,),
        in_specs=[pl.BlockSpec((512, DIM), lambda i: (i, 0)),
                  pl.BlockSpec((DIM, DIM), lambda i: (0, 0)),
                  pl.BlockSpec((1, DIM), lambda i: (0, 0))],
        out_specs=[pl.BlockSpec((512, DIM), lambda i: (i, 0)),
                   pl.BlockSpec((512, DIM), lambda i: (i, 0))],
        out_shape=[jax.ShapeDtypeStruct((N_EMBED, DIM), jnp.float32),
                   jax.ShapeDtypeStruct((N_EMBED, DIM), jnp.float32)],
        interpret=interpret,
    )(emb, w, b2d)


def _main_body(x_ref, cb_ref, s_ref, c2_ref, idx_ref, ent_ref, mse_ref):
    i = pl.program_id(0)
    x = x_ref[...]
    s = s_ref[...]

    iot = lax.broadcasted_iota(jnp.int32, (TM, KT), 1).astype(jnp.float32)
    jcol = lax.broadcasted_iota(jnp.int32, (KT, 1), 0)
    w_idx = jnp.concatenate(
        [(jcol >> 7).astype(jnp.bfloat16),
         (jcol & 127).astype(jnp.bfloat16),
         jnp.ones((KT, 1), jnp.bfloat16)], axis=1)

    def kstep(k, carry):
        m_run, z_run, t_run, bidx = carry
        cbk = cb_ref[pl.ds(k * KT, KT), :]
        c2k = c2_ref[:, pl.ds(k * KT, KT)]
        mm2 = lax.dot_general(x, cbk, (((1,), (1,)), ((), ())),
                              preferred_element_type=jnp.float32)
        l = mm2 - (s + c2k)
        tmax = jnp.max(l, axis=1, keepdims=True)
        eqf = (l == tmax).astype(jnp.bfloat16)
        sums = lax.dot_general(eqf, w_idx, (((1,), (0,)), ((), ())),
                               preferred_element_type=jnp.float32)
        cnt = sums[:, 2:3]

        def _fast():
            return (sums[:, 0:1] * 128.0 + sums[:, 1:2]).astype(jnp.int32)

        def _slow():
            masked = jnp.where(l == tmax, iot, jnp.float32(1e9))
            return jnp.min(masked, axis=1, keepdims=True).astype(jnp.int32)

        tidx = lax.cond(jnp.max(cnt) > 1.5, _slow, _fast) + k * KT
        m_new = jnp.maximum(m_run, tmax)
        alpha = jnp.exp(m_run - m_new)
        delta = m_new - m_run
        u = l - m_new
        e = jnp.exp(u)
        ew = e * u
        zt = jnp.sum(e, axis=1, keepdims=True)
        tt = jnp.sum(ew, axis=1, keepdims=True)
        z_new = z_run * alpha + zt
        t_new = (t_run - delta * z_run) * alpha + tt
        bidx = jnp.where(tmax > m_run, tidx, bidx)
        return (m_new, z_new, t_new, bidx)

    init = (jnp.full((TM, 1), jnp.finfo(jnp.float32).min, jnp.float32),
            jnp.zeros((TM, 1), jnp.float32),
            jnp.zeros((TM, 1), jnp.float32),
            jnp.zeros((TM, 1), jnp.int32))
    m_run, z_run, t_run, bidx = lax.fori_loop(0, NK, kstep, init)

    h = jnp.log(z_run) - t_run / z_run
    idx_ref[...] = bidx
    ent_s = jnp.sum(h).reshape(1, 1)
    mse_s = (-jnp.sum(m_run)).reshape(1, 1)

    @pl.when(i == 0)
    def _():
        ent_ref[...] = ent_s
        mse_ref[...] = mse_s

    @pl.when(i > 0)
    def _():
        ent_ref[...] += ent_s
        mse_ref[...] += mse_s


def _main_call(flatten, codebook2, s, c2row, *, interpret=False):
    return pl.pallas_call(
        _main_body,
        grid=(NTOK // TM,),
        in_specs=[pl.BlockSpec((TM, DIM), lambda i: (i, 0)),
                  pl.BlockSpec((N_EMBED, DIM), lambda i: (0, 0)),
                  pl.BlockSpec((TM, 1), lambda i: (i, 0)),
                  pl.BlockSpec((1, N_EMBED), lambda i: (0, 0))],
        out_specs=[pl.BlockSpec((TM, 1), lambda i: (i, 0)),
                   pl.BlockSpec((1, 1), lambda i: (0, 0)),
                   pl.BlockSpec((1, 1), lambda i: (0, 0))],
        out_shape=[jax.ShapeDtypeStruct((NTOK, 1), jnp.int32),
                   jax.ShapeDtypeStruct((1, 1), jnp.float32),
                   jax.ShapeDtypeStruct((1, 1), jnp.float32)],
        interpret=interpret,
    )(flatten, codebook2, s, c2row)


def _sc_body(cb_hbm, idx_hbm, zq_hbm, idx_v, rows_v, sem):
    wid = lax.axis_index("s") * SC_NC + lax.axis_index("c")
    base = wid * SC_BPW
    pltpu.sync_copy(idx_hbm.at[pl.ds(base, SC_BPW)], idx_v)
    pltpu.async_copy(cb_hbm.at[idx_v], rows_v, sem).wait()
    pltpu.sync_copy(rows_v, zq_hbm.at[pl.ds(base, SC_BPW)])


def _sc_call(codebook, indices):
    mesh = plsc.VectorSubcoreMesh(core_axis_name="c", subcore_axis_name="s")
    fn = pl.kernel(
        _sc_body,
        out_type=[jax.ShapeDtypeStruct((NTOK, DIM), jnp.float32)],
        mesh=mesh,
        scratch_types=[pltpu.VMEM((SC_BPW,), jnp.int32),
                       pltpu.VMEM((SC_BPW, DIM), jnp.float32),
                       pltpu.SemaphoreType.DMA],
    )
    return fn(codebook, indices)


def _final_body(ent_ref, mse_ref, idx_ref, out_ref):
    iot_hi = lax.broadcasted_iota(jnp.int32, (TM, 64), 1)
    iot_lo = lax.broadcasted_iota(jnp.int32, (TM, 128), 1)

    def hstep(t, acc):
        idxc = idx_ref[pl.ds(t * TM, TM), :]
        oh_hi = (iot_hi == (idxc >> 7)).astype(jnp.bfloat16)
        oh_lo = (iot_lo == (idxc & 127)).astype(jnp.bfloat16)
        return acc + lax.dot_general(oh_hi, oh_lo, (((0,), (0,)), ((), ())),
                                     preferred_element_type=jnp.float32)

    hist = lax.fori_loop(0, NTOK // TM, hstep,
                         jnp.zeros((64, 128), jnp.float32))
    avg = hist / float(NTOK) + EPS
    tue = (-jnp.sum(avg * jnp.log(avg))).reshape(1, 1)
    soft_entropy = ent_ref[...] / float(NTOK)
    sel = ENTROPY_PENALTY * (1.0 - soft_entropy / LOG_N)
    dl = mse_ref[...] / float(NTOK * DIM)
    vq = dl + BETA * dl
    tel = ENTROPY_PENALTY * (1.0 - tue / jnp.log(jnp.float32(N_EMBED)))
    out_ref[...] = vq + sel + tel


def _final_call(ent, mse, idx2, *, interpret=False):
    return pl.pallas_call(
        _final_body,
        in_specs=[pl.BlockSpec((1, 1), lambda: (0, 0)),
                  pl.BlockSpec((1, 1), lambda: (0, 0)),
                  pl.BlockSpec((NTOK, 1), lambda: (0, 0))],
        out_specs=pl.BlockSpec((1, 1), lambda: (0, 0)),
        out_shape=jax.ShapeDtypeStruct((1, 1), jnp.float32),
        interpret=interpret,
    )(ent, mse, idx2)


def kernel(input, emb_weight, proj_W, proj_b):
    Bs, Ts, C = input.shape
    flatten = input.reshape(-1, C)
    codebook, codebook2 = _codebook_call(emb_weight, proj_W, proj_b.reshape(1, C))
    s = jnp.sum(flatten ** 2, axis=1, keepdims=True)
    c2 = jnp.sum(codebook ** 2, axis=1)
    idx2, ent, mse = _main_call(flatten, codebook2, s, c2.reshape(1, N_EMBED))
    indices = idx2.reshape(NTOK)
    (z_q,) = _sc_call(codebook, indices)
    total = _final_call(ent, mse, idx2)[0, 0]
    return (z_q.reshape(Bs, Ts, C), total, indices)


# speedup vs baseline: 1.3197x; 1.3197x over previous
import math

import jax
import jax.numpy as jnp
from jax import lax
from jax.experimental import pallas as pl
from jax.experimental.pallas import tpu as pltpu
from jax.experimental.pallas import tpu_sc as plsc

DIM = 256
N_EMBED = 8192
NTOK = 9216
ENTROPY_PENALTY = 0.1
BETA = 0.25
EPS = 1e-05
LOG_N = math.log(float(N_EMBED))

TM = 512
KT = 1024
NK = N_EMBED // KT

SC_NC = 2
SC_NS = 16
SC_NW = SC_NC * SC_NS
SC_BPW = NTOK // SC_NW
HIST_PAD = N_EMBED + 16


def _codebook_body(emb_ref, w_ref, b_ref, out_ref, out2_ref):
    mm = lax.dot_general(emb_ref[...], w_ref[...], (((1,), (1,)), ((), ())),
                         preferred_element_type=jnp.float32)
    cb = mm + b_ref[...]
    out_ref[...] = cb
    out2_ref[...] = cb + cb


def _codebook_call(emb, w, b2d, *, interpret=False):
    return pl.pallas_call(
        _codebook_body,
        grid=(N_EMBED // 512,),
        in_specs=[pl.BlockSpec((512, DIM), lambda i: (i, 0)),
                  pl.BlockSpec((DIM, DIM), lambda i: (0, 0)),
                  pl.BlockSpec((1, DIM), lambda i: (0, 0))],
        out_specs=[pl.BlockSpec((512, DIM), lambda i: (i, 0)),
                   pl.BlockSpec((512, DIM), lambda i: (i, 0))],
        out_shape=[jax.ShapeDtypeStruct((N_EMBED, DIM), jnp.float32),
                   jax.ShapeDtypeStruct((N_EMBED, DIM), jnp.float32)],
        interpret=interpret,
    )(emb, w, b2d)


def _main_body(x_ref, cb_ref, s_ref, c2_ref, idx_ref, ent_ref, mse_ref):
    i = pl.program_id(0)
    x = x_ref[...]
    s = s_ref[...]

    iot = lax.broadcasted_iota(jnp.int32, (TM, KT), 1).astype(jnp.float32)

    def kstep(k, carry):
        m_run, z_run, t_run, bidx = carry
        cbk = cb_ref[pl.ds(k * KT, KT), :]
        c2k = c2_ref[:, pl.ds(k * KT, KT)]
        mm2 = lax.dot_general(x, cbk, (((1,), (1,)), ((), ())),
                              preferred_element_type=jnp.float32)
        l = mm2 - (s + c2k)
        tmax = jnp.max(l, axis=1, keepdims=True)
        masked = jnp.where(l == tmax, iot, jnp.float32(1e9))
        tidx = (jnp.min(masked, axis=1, keepdims=True)
                .astype(jnp.int32) + k * KT)
        m_new = jnp.maximum(m_run, tmax)
        alpha = jnp.exp(m_run - m_new)
        delta = m_new - m_run
        u = l - m_new
        e = jnp.exp(u)
        ew = e * u
        zt = jnp.sum(e, axis=1, keepdims=True)
        tt = jnp.sum(ew, axis=1, keepdims=True)
        z_new = z_run * alpha + zt
        t_new = (t_run - delta * z_run) * alpha + tt
        bidx = jnp.where(tmax > m_run, tidx, bidx)
        return (m_new, z_new, t_new, bidx)

    init = (jnp.full((TM, 1), jnp.finfo(jnp.float32).min, jnp.float32),
            jnp.zeros((TM, 1), jnp.float32),
            jnp.zeros((TM, 1), jnp.float32),
            jnp.zeros((TM, 1), jnp.int32))
    m_run, z_run, t_run, bidx = lax.fori_loop(0, NK, kstep, init)

    h = jnp.log(z_run) - t_run / z_run
    idx_ref[...] = bidx
    ent_s = jnp.sum(h).reshape(1, 1)
    mse_s = (-jnp.sum(m_run)).reshape(1, 1)

    @pl.when(i == 0)
    def _():
        ent_ref[...] = ent_s
        mse_ref[...] = mse_s

    @pl.when(i > 0)
    def _():
        ent_ref[...] += ent_s
        mse_ref[...] += mse_s


def _main_call(flatten, codebook2, s, c2row, *, interpret=False):
    return pl.pallas_call(
        _main_body,
        grid=(NTOK // TM,),
        in_specs=[pl.BlockSpec((TM, DIM), lambda i: (i, 0)),
                  pl.BlockSpec((N_EMBED, DIM), lambda i: (0, 0)),
                  pl.BlockSpec((TM, 1), lambda i: (i, 0)),
                  pl.BlockSpec((1, N_EMBED), lambda i: (0, 0))],
        out_specs=[pl.BlockSpec((TM, 1), lambda i: (i, 0)),
                   pl.BlockSpec((1, 1), lambda i: (0, 0)),
                   pl.BlockSpec((1, 1), lambda i: (0, 0))],
        out_shape=[jax.ShapeDtypeStruct((NTOK, 1), jnp.int32),
                   jax.ShapeDtypeStruct((1, 1), jnp.float32),
                   jax.ShapeDtypeStruct((1, 1), jnp.float32)],
        interpret=interpret,
    )(flatten, codebook2, s, c2row)


def _sc_body(cb_hbm, idx_hbm, zq_hbm, idx_v, rows_v, sem):
    wid = lax.axis_index("s") * SC_NC + lax.axis_index("c")
    base = wid * SC_BPW
    pltpu.sync_copy(idx_hbm.at[pl.ds(base, SC_BPW)], idx_v)
    pltpu.async_copy(cb_hbm.at[idx_v], rows_v, sem).wait()
    pltpu.sync_copy(rows_v, zq_hbm.at[pl.ds(base, SC_BPW)])


def _sc_call(codebook, indices):
    mesh = plsc.VectorSubcoreMesh(core_axis_name="c", subcore_axis_name="s")
    fn = pl.kernel(
        _sc_body,
        out_type=[jax.ShapeDtypeStruct((NTOK, DIM), jnp.float32)],
        mesh=mesh,
        scratch_types=[pltpu.VMEM((SC_BPW,), jnp.int32),
                       pltpu.VMEM((SC_BPW, DIM), jnp.float32),
                       pltpu.SemaphoreType.DMA],
    )
    return fn(codebook, indices)


def _final_body(ent_ref, mse_ref, idx_ref, out_ref):
    iot_hi = lax.broadcasted_iota(jnp.int32, (TM, 64), 1)
    iot_lo = lax.broadcasted_iota(jnp.int32, (TM, 128), 1)

    def hstep(t, acc):
        idxc = idx_ref[pl.ds(t * TM, TM), :]
        oh_hi = (iot_hi == (idxc >> 7)).astype(jnp.bfloat16)
        oh_lo = (iot_lo == (idxc & 127)).astype(jnp.bfloat16)
        return acc + lax.dot_general(oh_hi, oh_lo, (((0,), (0,)), ((), ())),
                                     preferred_element_type=jnp.float32)

    hist = lax.fori_loop(0, NTOK // TM, hstep,
                         jnp.zeros((64, 128), jnp.float32))
    avg = hist / float(NTOK) + EPS
    tue = (-jnp.sum(avg * jnp.log(avg))).reshape(1, 1)
    soft_entropy = ent_ref[...] / float(NTOK)
    sel = ENTROPY_PENALTY * (1.0 - soft_entropy / LOG_N)
    dl = mse_ref[...] / float(NTOK * DIM)
    vq = dl + BETA * dl
    tel = ENTROPY_PENALTY * (1.0 - tue / jnp.log(jnp.float32(N_EMBED)))
    out_ref[...] = vq + sel + tel


def _final_call(ent, mse, idx2, *, interpret=False):
    return pl.pallas_call(
        _final_body,
        in_specs=[pl.BlockSpec((1, 1), lambda: (0, 0)),
                  pl.BlockSpec((1, 1), lambda: (0, 0)),
                  pl.BlockSpec((NTOK, 1), lambda: (0, 0))],
        out_specs=pl.BlockSpec((1, 1), lambda: (0, 0)),
        out_shape=jax.ShapeDtypeStruct((1, 1), jnp.float32),
        interpret=interpret,
    )(ent, mse, idx2)


def kernel(input, emb_weight, proj_W, proj_b):
    Bs, Ts, C = input.shape
    flatten = input.reshape(-1, C)
    codebook, codebook2 = _codebook_call(emb_weight, proj_W, proj_b.reshape(1, C))
    s = jnp.sum(flatten ** 2, axis=1, keepdims=True)
    c2 = jnp.sum(codebook ** 2, axis=1)
    idx2, ent, mse = _main_call(flatten, codebook2, s, c2.reshape(1, N_EMBED))
    indices = idx2.reshape(NTOK)
    (z_q,) = _sc_call(codebook, indices)
    total = _final_call(ent, mse, idx2)[0, 0]
    return (z_q.reshape(Bs, Ts, C), total, indices)


# speedup vs baseline: 1.5207x; 1.1524x over previous
import math

import jax
import jax.numpy as jnp
from jax import lax
from jax.experimental import pallas as pl
from jax.experimental.pallas import tpu as pltpu
from jax.experimental.pallas import tpu_sc as plsc

DIM = 256
N_EMBED = 8192
NTOK = 9216
ENTROPY_PENALTY = 0.1
BETA = 0.25
EPS = 1e-05
LOG_N = math.log(float(N_EMBED))

TM = 768
KT = 2048
NK = N_EMBED // KT

SC_NC = 2
SC_NS = 16
SC_NW = SC_NC * SC_NS
SC_BPW = NTOK // SC_NW
HIST_PAD = N_EMBED + 16


def _codebook_body(emb_ref, w_ref, b_ref, out_ref, out2_ref):
    mm = lax.dot_general(emb_ref[...], w_ref[...], (((1,), (1,)), ((), ())),
                         preferred_element_type=jnp.float32)
    cb = mm + b_ref[...]
    out_ref[...] = cb
    out2_ref[...] = cb + cb


def _codebook_call(emb, w, b2d, *, interpret=False):
    return pl.pallas_call(
        _codebook_body,
        grid=(N_EMBED // 512,),
        in_specs=[pl.BlockSpec((512, DIM), lambda i: (i, 0)),
                  pl.BlockSpec((DIM, DIM), lambda i: (0, 0)),
                  pl.BlockSpec((1, DIM), lambda i: (0, 0))],
        out_specs=[pl.BlockSpec((512, DIM), lambda i: (i, 0)),
                   pl.BlockSpec((512, DIM), lambda i: (i, 0))],
        out_shape=[jax.ShapeDtypeStruct((N_EMBED, DIM), jnp.float32),
                   jax.ShapeDtypeStruct((N_EMBED, DIM), jnp.float32)],
        interpret=interpret,
    )(emb, w, b2d)


def _main_body(x_ref, cb_ref, s_ref, c2_ref, idx_ref, ent_ref, mse_ref):
    i = pl.program_id(0)
    x = x_ref[...]
    s = s_ref[...]

    iot = lax.broadcasted_iota(jnp.int32, (TM, KT), 1).astype(jnp.float32)

    def kstep(k, carry):
        m_run, z_run, t_run, bidx = carry
        cbk = cb_ref[pl.ds(k * KT, KT), :]
        c2k = c2_ref[:, pl.ds(k * KT, KT)]
        mm2 = lax.dot_general(x, cbk, (((1,), (1,)), ((), ())),
                              preferred_element_type=jnp.float32)
        l = mm2 - (s + c2k)
        tmax = jnp.max(l, axis=1, keepdims=True)
        masked = jnp.where(l == tmax, iot, jnp.float32(1e9))
        tidx = (jnp.min(masked, axis=1, keepdims=True)
                .astype(jnp.int32) + k * KT)
        m_new = jnp.maximum(m_run, tmax)
        alpha = jnp.exp(m_run - m_new)
        delta = m_new - m_run
        u = l - m_new
        e = jnp.exp(u)
        ew = e * u
        zt = jnp.sum(e, axis=1, keepdims=True)
        tt = jnp.sum(ew, axis=1, keepdims=True)
        z_new = z_run * alpha + zt
        t_new = (t_run - delta * z_run) * alpha + tt
        bidx = jnp.where(tmax > m_run, tidx, bidx)
        return (m_new, z_new, t_new, bidx)

    init = (jnp.full((TM, 1), jnp.finfo(jnp.float32).min, jnp.float32),
            jnp.zeros((TM, 1), jnp.float32),
            jnp.zeros((TM, 1), jnp.float32),
            jnp.zeros((TM, 1), jnp.int32))
    m_run, z_run, t_run, bidx = lax.fori_loop(0, NK, kstep, init)

    h = jnp.log(z_run) - t_run / z_run
    idx_ref[...] = bidx
    ent_s = jnp.sum(h).reshape(1, 1)
    mse_s = (-jnp.sum(m_run)).reshape(1, 1)

    @pl.when(i == 0)
    def _():
        ent_ref[...] = ent_s
        mse_ref[...] = mse_s

    @pl.when(i > 0)
    def _():
        ent_ref[...] += ent_s
        mse_ref[...] += mse_s


def _main_call(flatten, codebook2, s, c2row, *, interpret=False):
    return pl.pallas_call(
        _main_body,
        grid=(NTOK // TM,),
        in_specs=[pl.BlockSpec((TM, DIM), lambda i: (i, 0)),
                  pl.BlockSpec((N_EMBED, DIM), lambda i: (0, 0)),
                  pl.BlockSpec((TM, 1), lambda i: (i, 0)),
                  pl.BlockSpec((1, N_EMBED), lambda i: (0, 0))],
        out_specs=[pl.BlockSpec((TM, 1), lambda i: (i, 0)),
                   pl.BlockSpec((1, 1), lambda i: (0, 0)),
                   pl.BlockSpec((1, 1), lambda i: (0, 0))],
        out_shape=[jax.ShapeDtypeStruct((NTOK, 1), jnp.int32),
                   jax.ShapeDtypeStruct((1, 1), jnp.float32),
                   jax.ShapeDtypeStruct((1, 1), jnp.float32)],
        interpret=interpret,
    )(flatten, codebook2, s, c2row)


def _sc_body(cb_hbm, idx_hbm, zq_hbm, idx_v, rows_v, sem):
    wid = lax.axis_index("s") * SC_NC + lax.axis_index("c")
    base = wid * SC_BPW
    pltpu.sync_copy(idx_hbm.at[pl.ds(base, SC_BPW)], idx_v)
    pltpu.async_copy(cb_hbm.at[idx_v], rows_v, sem).wait()
    pltpu.sync_copy(rows_v, zq_hbm.at[pl.ds(base, SC_BPW)])


def _sc_call(codebook, indices):
    mesh = plsc.VectorSubcoreMesh(core_axis_name="c", subcore_axis_name="s")
    fn = pl.kernel(
        _sc_body,
        out_type=[jax.ShapeDtypeStruct((NTOK, DIM), jnp.float32)],
        mesh=mesh,
        scratch_types=[pltpu.VMEM((SC_BPW,), jnp.int32),
                       pltpu.VMEM((SC_BPW, DIM), jnp.float32),
                       pltpu.SemaphoreType.DMA],
    )
    return fn(codebook, indices)


def _final_body(ent_ref, mse_ref, idx_ref, out_ref):
    iot_hi = lax.broadcasted_iota(jnp.int32, (TM, 64), 1)
    iot_lo = lax.broadcasted_iota(jnp.int32, (TM, 128), 1)

    def hstep(t, acc):
        idxc = idx_ref[pl.ds(t * TM, TM), :]
        oh_hi = (iot_hi == (idxc >> 7)).astype(jnp.bfloat16)
        oh_lo = (iot_lo == (idxc & 127)).astype(jnp.bfloat16)
        return acc + lax.dot_general(oh_hi, oh_lo, (((0,), (0,)), ((), ())),
                                     preferred_element_type=jnp.float32)

    hist = lax.fori_loop(0, NTOK // TM, hstep,
                         jnp.zeros((64, 128), jnp.float32))
    avg = hist / float(NTOK) + EPS
    tue = (-jnp.sum(avg * jnp.log(avg))).reshape(1, 1)
    soft_entropy = ent_ref[...] / float(NTOK)
    sel = ENTROPY_PENALTY * (1.0 - soft_entropy / LOG_N)
    dl = mse_ref[...] / float(NTOK * DIM)
    vq = dl + BETA * dl
    tel = ENTROPY_PENALTY * (1.0 - tue / jnp.log(jnp.float32(N_EMBED)))
    out_ref[...] = vq + sel + tel


def _final_call(ent, mse, idx2, *, interpret=False):
    return pl.pallas_call(
        _final_body,
        in_specs=[pl.BlockSpec((1, 1), lambda: (0, 0)),
                  pl.BlockSpec((1, 1), lambda: (0, 0)),
                  pl.BlockSpec((NTOK, 1), lambda: (0, 0))],
        out_specs=pl.BlockSpec((1, 1), lambda: (0, 0)),
        out_shape=jax.ShapeDtypeStruct((1, 1), jnp.float32),
        interpret=interpret,
    )(ent, mse, idx2)


def kernel(input, emb_weight, proj_W, proj_b):
    Bs, Ts, C = input.shape
    flatten = input.reshape(-1, C)
    codebook, codebook2 = _codebook_call(emb_weight, proj_W, proj_b.reshape(1, C))
    s = jnp.sum(flatten ** 2, axis=1, keepdims=True)
    c2 = jnp.sum(codebook ** 2, axis=1)
    idx2, ent, mse = _main_call(flatten, codebook2, s, c2.reshape(1, N_EMBED))
    indices = idx2.reshape(NTOK)
    (z_q,) = _sc_call(codebook, indices)
    total = _final_call(ent, mse, idx2)[0, 0]
    return (z_q.reshape(Bs, Ts, C), total, indices)


# speedup vs baseline: 1.5383x; 1.0116x over previous
import math

import jax
import jax.numpy as jnp
from jax import lax
from jax.experimental import pallas as pl
from jax.experimental.pallas import tpu as pltpu
from jax.experimental.pallas import tpu_sc as plsc

DIM = 256
N_EMBED = 8192
NTOK = 9216
ENTROPY_PENALTY = 0.1
BETA = 0.25
EPS = 1e-05
LOG_N = math.log(float(N_EMBED))

TM = 1024
KT = 2048
NK = N_EMBED // KT

SC_NC = 2
SC_NS = 16
SC_NW = SC_NC * SC_NS
SC_BPW = NTOK // SC_NW
HIST_PAD = N_EMBED + 16


def _codebook_body(emb_ref, w_ref, b_ref, out_ref, out2_ref):
    mm = lax.dot_general(emb_ref[...], w_ref[...], (((1,), (1,)), ((), ())),
                         preferred_element_type=jnp.float32)
    cb = mm + b_ref[...]
    out_ref[...] = cb
    out2_ref[...] = cb + cb


def _codebook_call(emb, w, b2d, *, interpret=False):
    return pl.pallas_call(
        _codebook_body,
        grid=(N_EMBED // 512,),
        in_specs=[pl.BlockSpec((512, DIM), lambda i: (i, 0)),
                  pl.BlockSpec((DIM, DIM), lambda i: (0, 0)),
                  pl.BlockSpec((1, DIM), lambda i: (0, 0))],
        out_specs=[pl.BlockSpec((512, DIM), lambda i: (i, 0)),
                   pl.BlockSpec((512, DIM), lambda i: (i, 0))],
        out_shape=[jax.ShapeDtypeStruct((N_EMBED, DIM), jnp.float32),
                   jax.ShapeDtypeStruct((N_EMBED, DIM), jnp.float32)],
        interpret=interpret,
    )(emb, w, b2d)


def _main_body(x_ref, cb_ref, s_ref, c2_ref, idx_ref, ent_ref, mse_ref):
    i = pl.program_id(0)
    x = x_ref[...]
    s = s_ref[...]

    iot = lax.broadcasted_iota(jnp.int32, (TM, KT), 1).astype(jnp.float32)

    def kstep(k, carry):
        m_run, z_run, t_run, bidx = carry
        cbk = cb_ref[pl.ds(k * KT, KT), :]
        c2k = c2_ref[:, pl.ds(k * KT, KT)]
        mm2 = lax.dot_general(x, cbk, (((1,), (1,)), ((), ())),
                              preferred_element_type=jnp.float32)
        l = mm2 - (s + c2k)
        tmax = jnp.max(l, axis=1, keepdims=True)
        masked = jnp.where(l == tmax, iot, jnp.float32(1e9))
        tidx = (jnp.min(masked, axis=1, keepdims=True)
                .astype(jnp.int32) + k * KT)
        m_new = jnp.maximum(m_run, tmax)
        alpha = jnp.exp(m_run - m_new)
        delta = m_new - m_run
        u = l - m_new
        e = jnp.exp(u)
        ew = e * u
        zt = jnp.sum(e, axis=1, keepdims=True)
        tt = jnp.sum(ew, axis=1, keepdims=True)
        z_new = z_run * alpha + zt
        t_new = (t_run - delta * z_run) * alpha + tt
        bidx = jnp.where(tmax > m_run, tidx, bidx)
        return (m_new, z_new, t_new, bidx)

    init = (jnp.full((TM, 1), jnp.finfo(jnp.float32).min, jnp.float32),
            jnp.zeros((TM, 1), jnp.float32),
            jnp.zeros((TM, 1), jnp.float32),
            jnp.zeros((TM, 1), jnp.int32))
    m_run, z_run, t_run, bidx = lax.fori_loop(0, NK, kstep, init)

    h = jnp.log(z_run) - t_run / z_run
    idx_ref[...] = bidx
    ent_s = jnp.sum(h).reshape(1, 1)
    mse_s = (-jnp.sum(m_run)).reshape(1, 1)

    @pl.when(i == 0)
    def _():
        ent_ref[...] = ent_s
        mse_ref[...] = mse_s

    @pl.when(i > 0)
    def _():
        ent_ref[...] += ent_s
        mse_ref[...] += mse_s


def _main_call(flatten, codebook2, s, c2row, *, interpret=False):
    return pl.pallas_call(
        _main_body,
        grid=(NTOK // TM,),
        in_specs=[pl.BlockSpec((TM, DIM), lambda i: (i, 0)),
                  pl.BlockSpec((N_EMBED, DIM), lambda i: (0, 0)),
                  pl.BlockSpec((TM, 1), lambda i: (i, 0)),
                  pl.BlockSpec((1, N_EMBED), lambda i: (0, 0))],
        out_specs=[pl.BlockSpec((TM, 1), lambda i: (i, 0)),
                   pl.BlockSpec((1, 1), lambda i: (0, 0)),
                   pl.BlockSpec((1, 1), lambda i: (0, 0))],
        out_shape=[jax.ShapeDtypeStruct((NTOK, 1), jnp.int32),
                   jax.ShapeDtypeStruct((1, 1), jnp.float32),
                   jax.ShapeDtypeStruct((1, 1), jnp.float32)],
        interpret=interpret,
    )(flatten, codebook2, s, c2row)


def _sc_body(cb_hbm, idx_hbm, zq_hbm, idx_v, rows_v, sem):
    wid = lax.axis_index("s") * SC_NC + lax.axis_index("c")
    base = wid * SC_BPW
    pltpu.sync_copy(idx_hbm.at[pl.ds(base, SC_BPW)], idx_v)
    pltpu.async_copy(cb_hbm.at[idx_v], rows_v, sem).wait()
    pltpu.sync_copy(rows_v, zq_hbm.at[pl.ds(base, SC_BPW)])


def _sc_call(codebook, indices):
    mesh = plsc.VectorSubcoreMesh(core_axis_name="c", subcore_axis_name="s")
    fn = pl.kernel(
        _sc_body,
        out_type=[jax.ShapeDtypeStruct((NTOK, DIM), jnp.float32)],
        mesh=mesh,
        scratch_types=[pltpu.VMEM((SC_BPW,), jnp.int32),
                       pltpu.VMEM((SC_BPW, DIM), jnp.float32),
                       pltpu.SemaphoreType.DMA],
    )
    return fn(codebook, indices)


def _final_body(ent_ref, mse_ref, idx_ref, out_ref):
    iot_hi = lax.broadcasted_iota(jnp.int32, (TM, 64), 1)
    iot_lo = lax.broadcasted_iota(jnp.int32, (TM, 128), 1)

    def hstep(t, acc):
        idxc = idx_ref[pl.ds(t * TM, TM), :]
        oh_hi = (iot_hi == (idxc >> 7)).astype(jnp.bfloat16)
        oh_lo = (iot_lo == (idxc & 127)).astype(jnp.bfloat16)
        return acc + lax.dot_general(oh_hi, oh_lo, (((0,), (0,)), ((), ())),
                                     preferred_element_type=jnp.float32)

    hist = lax.fori_loop(0, NTOK // TM, hstep,
                         jnp.zeros((64, 128), jnp.float32))
    avg = hist / float(NTOK) + EPS
    tue = (-jnp.sum(avg * jnp.log(avg))).reshape(1, 1)
    soft_entropy = ent_ref[...] / float(NTOK)
    sel = ENTROPY_PENALTY * (1.0 - soft_entropy / LOG_N)
    dl = mse_ref[...] / float(NTOK * DIM)
    vq = dl + BETA * dl
    tel = ENTROPY_PENALTY * (1.0 - tue / jnp.log(jnp.float32(N_EMBED)))
    out_ref[...] = vq + sel + tel


def _final_call(ent, mse, idx2, *, interpret=False):
    return pl.pallas_call(
        _final_body,
        in_specs=[pl.BlockSpec((1, 1), lambda: (0, 0)),
                  pl.BlockSpec((1, 1), lambda: (0, 0)),
                  pl.BlockSpec((NTOK, 1), lambda: (0, 0))],
        out_specs=pl.BlockSpec((1, 1), lambda: (0, 0)),
        out_shape=jax.ShapeDtypeStruct((1, 1), jnp.float32),
        interpret=interpret,
    )(ent, mse, idx2)


def kernel(input, emb_weight, proj_W, proj_b):
    Bs, Ts, C = input.shape
    flatten = input.reshape(-1, C)
    codebook, codebook2 = _codebook_call(emb_weight, proj_W, proj_b.reshape(1, C))
    s = jnp.sum(flatten ** 2, axis=1, keepdims=True)
    c2 = jnp.sum(codebook ** 2, axis=1)
    idx2, ent, mse = _main_call(flatten, codebook2, s, c2.reshape(1, N_EMBED))
    indices = idx2.reshape(NTOK)
    (z_q,) = _sc_call(codebook, indices)
    total = _final_call(ent, mse, idx2)[0, 0]
    return (z_q.reshape(Bs, Ts, C), total, indices)


# speedup vs baseline: 1.5415x; 1.0021x over previous
import math

import jax
import jax.numpy as jnp
from jax import lax
from jax.experimental import pallas as pl
from jax.experimental.pallas import tpu as pltpu
from jax.experimental.pallas import tpu_sc as plsc

DIM = 256
N_EMBED = 8192
NTOK = 9216
ENTROPY_PENALTY = 0.1
BETA = 0.25
EPS = 1e-05
LOG_N = math.log(float(N_EMBED))

TM = 1152
KT = 2048
NK = N_EMBED // KT

SC_NC = 2
SC_NS = 16
SC_NW = SC_NC * SC_NS
SC_BPW = NTOK // SC_NW
HIST_PAD = N_EMBED + 16


def _codebook_body(emb_ref, w_ref, b_ref, out_ref, out2_ref):
    mm = lax.dot_general(emb_ref[...], w_ref[...], (((1,), (1,)), ((), ())),
                         preferred_element_type=jnp.float32)
    cb = mm + b_ref[...]
    out_ref[...] = cb
    out2_ref[...] = cb + cb


def _codebook_call(emb, w, b2d, *, interpret=False):
    return pl.pallas_call(
        _codebook_body,
        grid=(N_EMBED // 512,),
        in_specs=[pl.BlockSpec((512, DIM), lambda i: (i, 0)),
                  pl.BlockSpec((DIM, DIM), lambda i: (0, 0)),
                  pl.BlockSpec((1, DIM), lambda i: (0, 0))],
        out_specs=[pl.BlockSpec((512, DIM), lambda i: (i, 0)),
                   pl.BlockSpec((512, DIM), lambda i: (i, 0))],
        out_shape=[jax.ShapeDtypeStruct((N_EMBED, DIM), jnp.float32),
                   jax.ShapeDtypeStruct((N_EMBED, DIM), jnp.float32)],
        interpret=interpret,
    )(emb, w, b2d)


def _main_body(x_ref, cb_ref, s_ref, c2_ref, idx_ref, ent_ref, mse_ref):
    i = pl.program_id(0)
    x = x_ref[...]
    s = s_ref[...]

    iot = lax.broadcasted_iota(jnp.int32, (TM, KT), 1).astype(jnp.float32)

    def kstep(k, carry):
        m_run, z_run, t_run, bidx = carry
        cbk = cb_ref[pl.ds(k * KT, KT), :]
        c2k = c2_ref[:, pl.ds(k * KT, KT)]
        mm2 = lax.dot_general(x, cbk, (((1,), (1,)), ((), ())),
                              preferred_element_type=jnp.float32)
        l = mm2 - (s + c2k)
        tmax = jnp.max(l, axis=1, keepdims=True)
        masked = jnp.where(l == tmax, iot, jnp.float32(1e9))
        tidx = (jnp.min(masked, axis=1, keepdims=True)
                .astype(jnp.int32) + k * KT)
        m_new = jnp.maximum(m_run, tmax)
        alpha = jnp.exp(m_run - m_new)
        delta = m_new - m_run
        u = l - m_new
        e = jnp.exp(u)
        ew = e * u
        zt = jnp.sum(e, axis=1, keepdims=True)
        tt = jnp.sum(ew, axis=1, keepdims=True)
        z_new = z_run * alpha + zt
        t_new = (t_run - delta * z_run) * alpha + tt
        bidx = jnp.where(tmax > m_run, tidx, bidx)
        return (m_new, z_new, t_new, bidx)

    init = (jnp.full((TM, 1), jnp.finfo(jnp.float32).min, jnp.float32),
            jnp.zeros((TM, 1), jnp.float32),
            jnp.zeros((TM, 1), jnp.float32),
            jnp.zeros((TM, 1), jnp.int32))
    m_run, z_run, t_run, bidx = lax.fori_loop(0, NK, kstep, init)

    h = jnp.log(z_run) - t_run / z_run
    idx_ref[...] = bidx
    ent_s = jnp.sum(h).reshape(1, 1)
    mse_s = (-jnp.sum(m_run)).reshape(1, 1)

    @pl.when(i == 0)
    def _():
        ent_ref[...] = ent_s
        mse_ref[...] = mse_s

    @pl.when(i > 0)
    def _():
        ent_ref[...] += ent_s
        mse_ref[...] += mse_s


def _main_call(flatten, codebook2, s, c2row, *, interpret=False):
    return pl.pallas_call(
        _main_body,
        grid=(NTOK // TM,),
        in_specs=[pl.BlockSpec((TM, DIM), lambda i: (i, 0)),
                  pl.BlockSpec((N_EMBED, DIM), lambda i: (0, 0)),
                  pl.BlockSpec((TM, 1), lambda i: (i, 0)),
                  pl.BlockSpec((1, N_EMBED), lambda i: (0, 0))],
        out_specs=[pl.BlockSpec((TM, 1), lambda i: (i, 0)),
                   pl.BlockSpec((1, 1), lambda i: (0, 0)),
                   pl.BlockSpec((1, 1), lambda i: (0, 0))],
        out_shape=[jax.ShapeDtypeStruct((NTOK, 1), jnp.int32),
                   jax.ShapeDtypeStruct((1, 1), jnp.float32),
                   jax.ShapeDtypeStruct((1, 1), jnp.float32)],
        interpret=interpret,
    )(flatten, codebook2, s, c2row)


def _sc_body(cb_hbm, idx_hbm, zq_hbm, idx_v, rows_v, sem):
    wid = lax.axis_index("s") * SC_NC + lax.axis_index("c")
    base = wid * SC_BPW
    pltpu.sync_copy(idx_hbm.at[pl.ds(base, SC_BPW)], idx_v)
    pltpu.async_copy(cb_hbm.at[idx_v], rows_v, sem).wait()
    pltpu.sync_copy(rows_v, zq_hbm.at[pl.ds(base, SC_BPW)])


def _sc_call(codebook, indices):
    mesh = plsc.VectorSubcoreMesh(core_axis_name="c", subcore_axis_name="s")
    fn = pl.kernel(
        _sc_body,
        out_type=[jax.ShapeDtypeStruct((NTOK, DIM), jnp.float32)],
        mesh=mesh,
        scratch_types=[pltpu.VMEM((SC_BPW,), jnp.int32),
                       pltpu.VMEM((SC_BPW, DIM), jnp.float32),
                       pltpu.SemaphoreType.DMA],
    )
    return fn(codebook, indices)


def _final_body(ent_ref, mse_ref, idx_ref, out_ref):
    iot_hi = lax.broadcasted_iota(jnp.int32, (TM, 64), 1)
    iot_lo = lax.broadcasted_iota(jnp.int32, (TM, 128), 1)

    def hstep(t, acc):
        idxc = idx_ref[pl.ds(t * TM, TM), :]
        oh_hi = (iot_hi == (idxc >> 7)).astype(jnp.bfloat16)
        oh_lo = (iot_lo == (idxc & 127)).astype(jnp.bfloat16)
        return acc + lax.dot_general(oh_hi, oh_lo, (((0,), (0,)), ((), ())),
                                     preferred_element_type=jnp.float32)

    hist = lax.fori_loop(0, NTOK // TM, hstep,
                         jnp.zeros((64, 128), jnp.float32))
    avg = hist / float(NTOK) + EPS
    tue = (-jnp.sum(avg * jnp.log(avg))).reshape(1, 1)
    soft_entropy = ent_ref[...] / float(NTOK)
    sel = ENTROPY_PENALTY * (1.0 - soft_entropy / LOG_N)
    dl = mse_ref[...] / float(NTOK * DIM)
    vq = dl + BETA * dl
    tel = ENTROPY_PENALTY * (1.0 - tue / jnp.log(jnp.float32(N_EMBED)))
    out_ref[...] = vq + sel + tel


def _final_call(ent, mse, idx2, *, interpret=False):
    return pl.pallas_call(
        _final_body,
        in_specs=[pl.BlockSpec((1, 1), lambda: (0, 0)),
                  pl.BlockSpec((1, 1), lambda: (0, 0)),
                  pl.BlockSpec((NTOK, 1), lambda: (0, 0))],
        out_specs=pl.BlockSpec((1, 1), lambda: (0, 0)),
        out_shape=jax.ShapeDtypeStruct((1, 1), jnp.float32),
        interpret=interpret,
    )(ent, mse, idx2)


def kernel(input, emb_weight, proj_W, proj_b):
    Bs, Ts, C = input.shape
    flatten = input.reshape(-1, C)
    codebook, codebook2 = _codebook_call(emb_weight, proj_W, proj_b.reshape(1, C))
    s = jnp.sum(flatten ** 2, axis=1, keepdims=True)
    c2 = jnp.sum(codebook ** 2, axis=1)
    idx2, ent, mse = _main_call(flatten, codebook2, s, c2.reshape(1, N_EMBED))
    indices = idx2.reshape(NTOK)
    (z_q,) = _sc_call(codebook, indices)
    total = _final_call(ent, mse, idx2)[0, 0]
    return (z_q.reshape(Bs, Ts, C), total, indices)
